# speedup vs baseline: 4.0071x; 4.0071x over previous
import jax
import jax.numpy as jnp
from jax import lax
from jax.experimental import pallas as pl

_R = 2048
_STYLE_DIM = 128
_VPAD = 32


def _tc_body(idx_ref, tab_ref, out_ref):
    idx = idx_ref[0, 0, :]
    iota = lax.broadcasted_iota(jnp.int32, (_VPAD, _R), 0)
    onehot = (iota == idx[None, :]).astype(jnp.float32)
    out_ref[0] = lax.dot_general(
        onehot, tab_ref[...],
        (((0,), (0,)), ((), ())),
        preferred_element_type=jnp.float32,
    )


def kernel(indices, table):
    B = indices.size
    nblk = B // _R
    idx = indices.reshape(nblk, 1, _R).astype(jnp.int32)
    tab = jnp.zeros((_VPAD, _STYLE_DIM), table.dtype).at[:table.shape[0]].set(table)
    out = pl.pallas_call(
        _tc_body,
        grid=(nblk,),
        in_specs=[
            pl.BlockSpec((1, 1, _R), lambda i: (i, 0, 0)),
            pl.BlockSpec((_VPAD, _STYLE_DIM), lambda i: (0, 0)),
        ],
        out_specs=pl.BlockSpec((1, _R, _STYLE_DIM), lambda i: (i, 0, 0)),
        out_shape=jax.ShapeDtypeStruct((nblk, _R, _STYLE_DIM), jnp.float32),
    )(idx, tab)
    return out.reshape(indices.shape + (_STYLE_DIM,))


# speedup vs baseline: 6.1866x; 1.5439x over previous
import jax
import jax.numpy as jnp
from jax import lax
from jax.experimental import pallas as pl

_RB = 16
_T = 100
_STYLE_DIM = 128
_VPAD = 32


def _tc_body(idx_ref, tab_ref, out_ref):
    tab = tab_ref[...]
    for i in range(_RB):
        idx = idx_ref[i, :]
        iota = lax.broadcasted_iota(jnp.int32, (_VPAD, _T), 0)
        onehot = (iota == idx[None, :]).astype(jnp.float32)
        out_ref[i] = lax.dot_general(
            onehot, tab,
            (((0,), (0,)), ((), ())),
            preferred_element_type=jnp.float32,
        )


def kernel(indices, table):
    n = indices.shape[0]
    nblk = n // _RB
    idx = indices.astype(jnp.int32)
    tab = jnp.zeros((_VPAD, _STYLE_DIM), table.dtype).at[:table.shape[0]].set(table)
    return pl.pallas_call(
        _tc_body,
        grid=(nblk,),
        in_specs=[
            pl.BlockSpec((_RB, _T), lambda i: (i, 0)),
            pl.BlockSpec((_VPAD, _STYLE_DIM), lambda i: (0, 0)),
        ],
        out_specs=pl.BlockSpec((_RB, _T, _STYLE_DIM), lambda i: (i, 0, 0)),
        out_shape=jax.ShapeDtypeStruct((n, _T, _STYLE_DIM), jnp.float32),
    )(idx, tab)


# speedup vs baseline: 8.9770x; 1.4510x over previous
import jax
import jax.numpy as jnp
from jax import lax
from jax.experimental import pallas as pl

_RB = 64
_T = 100
_STYLE_DIM = 128
_VPAD = 32


def _tc_body(idx_ref, tab_ref, out_ref):
    tab = tab_ref[...]
    for i in range(_RB):
        idx = idx_ref[i, :]
        iota = lax.broadcasted_iota(jnp.int32, (_VPAD, _T), 0)
        onehot = (iota == idx[None, :]).astype(jnp.float32)
        out_ref[i] = lax.dot_general(
            onehot, tab,
            (((0,), (0,)), ((), ())),
            preferred_element_type=jnp.float32,
        )


def kernel(indices, table):
    n = indices.shape[0]
    nblk = n // _RB
    idx = indices.astype(jnp.int32)
    tab = jnp.zeros((_VPAD, _STYLE_DIM), table.dtype).at[:table.shape[0]].set(table)
    return pl.pallas_call(
        _tc_body,
        grid=(nblk,),
        in_specs=[
            pl.BlockSpec((_RB, _T), lambda i: (i, 0)),
            pl.BlockSpec((_VPAD, _STYLE_DIM), lambda i: (0, 0)),
        ],
        out_specs=pl.BlockSpec((_RB, _T, _STYLE_DIM), lambda i: (i, 0, 0)),
        out_shape=jax.ShapeDtypeStruct((n, _T, _STYLE_DIM), jnp.float32),
    )(idx, tab)


# speedup vs baseline: 9.6355x; 1.0733x over previous
import jax
import jax.numpy as jnp
from jax import lax
from jax.experimental import pallas as pl

_RB = 128
_T = 100
_STYLE_DIM = 128
_VPAD = 32


def _tc_body(idx_ref, tab_ref, out_ref):
    tab = tab_ref[...]
    for i in range(_RB):
        idx = idx_ref[i, :]
        iota = lax.broadcasted_iota(jnp.int32, (_VPAD, _T), 0)
        onehot = (iota == idx[None, :]).astype(jnp.float32)
        out_ref[i] = lax.dot_general(
            onehot, tab,
            (((0,), (0,)), ((), ())),
            preferred_element_type=jnp.float32,
        )


def kernel(indices, table):
    n = indices.shape[0]
    nblk = n // _RB
    idx = indices.astype(jnp.int32)
    tab = jnp.zeros((_VPAD, _STYLE_DIM), table.dtype).at[:table.shape[0]].set(table)
    return pl.pallas_call(
        _tc_body,
        grid=(nblk,),
        in_specs=[
            pl.BlockSpec((_RB, _T), lambda i: (i, 0)),
            pl.BlockSpec((_VPAD, _STYLE_DIM), lambda i: (0, 0)),
        ],
        out_specs=pl.BlockSpec((_RB, _T, _STYLE_DIM), lambda i: (i, 0, 0)),
        out_shape=jax.ShapeDtypeStruct((n, _T, _STYLE_DIM), jnp.float32),
    )(idx, tab)


# speedup vs baseline: 9.9462x; 1.0323x over previous
import jax
import jax.numpy as jnp
from jax import lax
from jax.experimental import pallas as pl

_RB = 256
_T = 100
_STYLE_DIM = 128
_VPAD = 32


def _tc_body(idx_ref, tab_ref, out_ref):
    tab = tab_ref[...]
    for i in range(_RB):
        idx = idx_ref[i, :]
        iota = lax.broadcasted_iota(jnp.int32, (_VPAD, _T), 0)
        onehot = (iota == idx[None, :]).astype(jnp.float32)
        out_ref[i] = lax.dot_general(
            onehot, tab,
            (((0,), (0,)), ((), ())),
            preferred_element_type=jnp.float32,
        )


def kernel(indices, table):
    n = indices.shape[0]
    nblk = n // _RB
    idx = indices.astype(jnp.int32)
    tab = jnp.zeros((_VPAD, _STYLE_DIM), table.dtype).at[:table.shape[0]].set(table)
    return pl.pallas_call(
        _tc_body,
        grid=(nblk,),
        in_specs=[
            pl.BlockSpec((_RB, _T), lambda i: (i, 0)),
            pl.BlockSpec((_VPAD, _STYLE_DIM), lambda i: (0, 0)),
        ],
        out_specs=pl.BlockSpec((_RB, _T, _STYLE_DIM), lambda i: (i, 0, 0)),
        out_shape=jax.ShapeDtypeStruct((n, _T, _STYLE_DIM), jnp.float32),
    )(idx, tab)
